# baseline (speedup 1.0000x reference)
import dataclasses

import jax
import jax.numpy as jnp
from jax import lax
from jax.experimental import pallas as pl
from jax.experimental.pallas import tpu as pltpu
from jax.experimental.pallas import tpu_sc as plsc

_N = 4096
_MAXDEG = 64

_NT = 2560
_BR = 512

_NSC = _N - _NT
_L = 16
_NC, _NS = 2, 16
_NW = _NC * _NS
_RPW = _NSC // _NW
_RBLK = 4
_NBLK = _RPW // _RBLK
_UNROLL = 8


def _tc_body(rank_ref, c_ref, mask_ref, out_ref):
    i = pl.program_id(0)
    r = rank_ref[pl.ds(i * _BR, _BR)]
    rc = jnp.minimum(r, _MAXDEG - 1)
    g = jnp.full((_BR,), c_ref[0], dtype=jnp.float32)
    for k in range(1, _MAXDEG):
        g = jnp.where(rc == k, c_ref[k], g)
    out_ref[...] = g[:, None] * mask_ref[...]


def _tc_rows(rank, c, sparse_mask):
    return pl.pallas_call(
        _tc_body,
        grid=(_NT // _BR,),
        in_specs=[
            pl.BlockSpec((_N,), lambda i: (0,)),
            pl.BlockSpec(memory_space=pltpu.SMEM),
            pl.BlockSpec((_BR, _N), lambda i: (i, 0)),
        ],
        out_specs=pl.BlockSpec((_BR, _N), lambda i: (i, 0)),
        out_shape=jax.ShapeDtypeStruct((_NT, _N), jnp.float32),
    )(rank, c, sparse_mask)


def _sc_body(rank_hbm, c_hbm, mask_hbm, out_hbm,
             c_v, rank_v, g_v, in0, in1, out0, out1,
             sem_in0, sem_in1, sem_out0, sem_out1):
    wid = lax.axis_index("s") * _NC + lax.axis_index("c")
    base = _NT + wid * _RPW
    obase = wid * _RPW

    pltpu.sync_copy(c_hbm, c_v)
    pltpu.sync_copy(rank_hbm.at[pl.ds(base, _RPW)], rank_v)

    @pl.loop(0, _RPW, step=_L)
    def _(i):
        rc = jnp.minimum(rank_v[pl.ds(i, _L)], _MAXDEG - 1)
        g_v[pl.ds(i, _L)] = plsc.load_gather(c_v, [rc])

    sem_in = (sem_in0, sem_in1)
    sem_out = (sem_out0, sem_out1)
    in_bufs = (in0, in1)
    out_bufs = (out0, out1)

    def mrows(blk):
        return pl.ds(base + blk * _RBLK, _RBLK)

    def orows(blk):
        return pl.ds(obase + blk * _RBLK, _RBLK)

    pltpu.async_copy(mask_hbm.at[mrows(0)], in_bufs[0], sem_in[0])
    pltpu.async_copy(mask_hbm.at[mrows(1)], in_bufs[1], sem_in[1])

    @pl.loop(0, _NBLK, step=2)
    def _(blk0):
        for b in (0, 1):
            blk = blk0 + b
            pltpu.make_async_copy(mask_hbm.at[mrows(blk)], in_bufs[b],
                                  sem_in[b]).wait()

            @pl.when(blk >= 2)
            def _():
                pltpu.make_async_copy(out_bufs[b], out_hbm.at[orows(blk)],
                                      sem_out[b]).wait()

            for row in range(_RBLK):
                ridx = blk * _RBLK + row
                gvec = plsc.load_gather(
                    g_v, [jnp.full((_L,), ridx, jnp.int32)])
                src = in_bufs[b]
                dst = out_bufs[b]

                @pl.loop(0, _N, step=_L * _UNROLL)
                def _(c0):
                    for u in range(_UNROLL):
                        sl = pl.ds(c0 + u * _L, _L)
                        dst[row, sl] = gvec * src[row, sl]

            pltpu.async_copy(out_bufs[b], out_hbm.at[orows(blk)], sem_out[b])

            @pl.when(blk + 2 < _NBLK)
            def _():
                pltpu.async_copy(mask_hbm.at[mrows(blk + 2)], in_bufs[b],
                                 sem_in[b])

    for b in (0, 1):
        blk = _NBLK - 2 + b
        pltpu.make_async_copy(out_bufs[b], out_hbm.at[orows(blk)],
                              sem_out[b]).wait()


def _sc_rows(rank, c, sparse_mask):
    mesh = plsc.VectorSubcoreMesh(core_axis_name="c", subcore_axis_name="s")
    cp = pltpu.CompilerParams()
    if "needs_layout_passes" in pltpu.CompilerParams.__dataclass_fields__:
        cp = dataclasses.replace(cp, needs_layout_passes=False)
    kern = pl.kernel(
        _sc_body,
        out_type=jax.ShapeDtypeStruct((_NSC, _N), jnp.float32),
        mesh=mesh,
        scratch_types=[
            pltpu.VMEM((_MAXDEG,), jnp.float32),
            pltpu.VMEM((_RPW,), jnp.int32),
            pltpu.VMEM((_RPW,), jnp.float32),
            pltpu.VMEM((_RBLK, _N), jnp.float32),
            pltpu.VMEM((_RBLK, _N), jnp.float32),
            pltpu.VMEM((_RBLK, _N), jnp.float32),
            pltpu.VMEM((_RBLK, _N), jnp.float32),
            pltpu.SemaphoreType.DMA,
            pltpu.SemaphoreType.DMA,
            pltpu.SemaphoreType.DMA,
            pltpu.SemaphoreType.DMA,
        ],
        compiler_params=cp,
    )
    return kern(rank, c, sparse_mask)


def kernel(x, rank, sparse_mask, c):
    del x
    top = _tc_rows(rank, c, sparse_mask)
    bottom = _sc_rows(rank, c, sparse_mask)
    return jnp.concatenate([top, bottom], axis=0)


# speedup vs baseline: 2.3307x; 2.3307x over previous
import jax
import jax.numpy as jnp
from jax.experimental import pallas as pl
from jax.experimental.pallas import tpu as pltpu

_N = 4096
_MAXDEG = 64
_BR = 512


def _row_scale_kernel(rank_ref, c_ref, mask_ref, out_ref):
    i = pl.program_id(0)
    r = rank_ref[pl.ds(i * _BR, _BR)]
    rc = jnp.minimum(r, _MAXDEG - 1)
    g = jnp.full((_BR,), c_ref[0], dtype=jnp.float32)
    for k in range(1, _MAXDEG):
        g = jnp.where(rc == k, c_ref[k], g)
    out_ref[...] = g[:, None] * mask_ref[...]


def kernel(x, rank, sparse_mask, c):
    del x
    grid = _N // _BR
    return pl.pallas_call(
        _row_scale_kernel,
        grid=(grid, 2),
        in_specs=[
            pl.BlockSpec((_N,), lambda i, j: (0,)),
            pl.BlockSpec(memory_space=pltpu.SMEM),
            pl.BlockSpec((_BR, _N // 2), lambda i, j: (i, j)),
        ],
        out_specs=pl.BlockSpec((_BR, _N // 2), lambda i, j: (i, j)),
        out_shape=jax.ShapeDtypeStruct((_N, _N), jnp.float32),
        compiler_params=pltpu.CompilerParams(
            dimension_semantics=("parallel", "parallel"),
        ),
    )(rank, c, sparse_mask)


# speedup vs baseline: 2.4386x; 1.0463x over previous
import jax
import jax.numpy as jnp
from jax.experimental import pallas as pl
from jax.experimental.pallas import tpu as pltpu

_N = 4096
_MAXDEG = 64
_BR = 512


def _row_scale_kernel(rank_ref, c_ref, mask_ref, out_ref):
    i = pl.program_id(0)
    r = rank_ref[pl.ds(i * _BR, _BR)]
    rc = jnp.minimum(r, _MAXDEG - 1)
    bits = [((rc >> b) & 1) == 1 for b in range(6)]
    vals = [c_ref[k] for k in range(_MAXDEG)]
    for b in range(6):
        vals = [jnp.where(bits[b], hi, lo)
                for lo, hi in zip(vals[0::2], vals[1::2])]
    g = vals[0]
    out_ref[...] = g[:, None] * mask_ref[...]


def kernel(x, rank, sparse_mask, c):
    del x
    grid = _N // _BR
    return pl.pallas_call(
        _row_scale_kernel,
        grid=(grid,),
        in_specs=[
            pl.BlockSpec((_N,), lambda i: (0,)),
            pl.BlockSpec(memory_space=pltpu.SMEM),
            pl.BlockSpec((_BR, _N), lambda i: (i, 0)),
        ],
        out_specs=pl.BlockSpec((_BR, _N), lambda i: (i, 0)),
        out_shape=jax.ShapeDtypeStruct((_N, _N), jnp.float32),
        compiler_params=pltpu.CompilerParams(
            dimension_semantics=("arbitrary",),
        ),
    )(rank, c, sparse_mask)
